# baseline (speedup 1.0000x reference)
import jax
import jax.numpy as jnp
from jax import lax
from jax.experimental import pallas as pl

NUM_TOKENS = 8192
CODE_DIM = 256
FEAT_DIM = 192
COMMITMENT = 0.25
N_FLAT = 8 * 56 * 56


def _conv(x, w, b, stride):
    y = lax.conv_general_dilated(x, w, (stride, stride), 'SAME',
                                 dimension_numbers=('NCHW', 'OIHW', 'NCHW'))
    return y + b[None, :, None, None]


def _conv_t(x, w, b, stride):
    y = lax.conv_transpose(x, w, (stride, stride), 'SAME',
                           dimension_numbers=('NCHW', 'OIHW', 'NCHW'))
    return y + b[None, :, None, None]


def _proj_body(tok_ref, wout_ref, b_ref, out_ref):
    out_ref[...] = lax.dot_general(tok_ref[...], wout_ref[...],
                                   (((1,), (1,)), ((), ())),
                                   preferred_element_type=jnp.float32
                                   ) + b_ref[...]


def _proj_out_mm(tokens_flat, wout2d, b2d):
    return pl.pallas_call(
        _proj_body,
        grid=(N_FLAT // 512,),
        in_specs=[pl.BlockSpec((512, CODE_DIM), lambda i: (i, 0)),
                  pl.BlockSpec((FEAT_DIM, CODE_DIM), lambda i: (0, 0)),
                  pl.BlockSpec((1, FEAT_DIM), lambda i: (0, 0))],
        out_specs=pl.BlockSpec((512, FEAT_DIM), lambda i: (i, 0)),
        out_shape=jax.ShapeDtypeStruct((N_FLAT, FEAT_DIM), jnp.float32),
    )(tokens_flat, wout2d, b2d)


def kernel(images, enc_w1, enc_b1, enc_w2, enc_b2, proj_in_w, proj_in_b,
           proj_out_w, proj_out_b, dec_w1, dec_b1, dec_w2, dec_b2, codebook):
    features = jax.nn.relu(_conv(images, enc_w1, enc_b1, 2))
    features = jax.nn.relu(_conv(features, enc_w2, enc_b2, 2))

    x = _conv(features, proj_in_w, proj_in_b, 1)
    inputs_nhwc = jnp.transpose(x, (0, 2, 3, 1))
    flat = inputs_nhwc.reshape(-1, CODE_DIM)
    distances = (jnp.sum(flat ** 2, axis=1, keepdims=True)
                 + jnp.sum(codebook ** 2, axis=1)
                 - 2.0 * jnp.matmul(flat, codebook.T))
    labels = jnp.argmin(distances, axis=1)

    encodings = jax.nn.one_hot(labels, NUM_TOKENS, dtype=flat.dtype)
    quantized = jnp.matmul(encodings, codebook).reshape(inputs_nhwc.shape)
    counts = jnp.bincount(labels, length=NUM_TOKENS)
    quantized = inputs_nhwc + lax.stop_gradient(quantized - inputs_nhwc)
    tokens_flat = quantized.reshape(-1, CODE_DIM)

    proj_flat = _proj_out_mm(tokens_flat,
                             proj_out_w.reshape(FEAT_DIM, CODE_DIM),
                             proj_out_b.reshape(1, FEAT_DIM))
    projected_tokens = jnp.transpose(
        proj_flat.reshape(8, 56, 56, FEAT_DIM), (0, 3, 1, 2))

    recon = jax.nn.relu(_conv_t(projected_tokens, dec_w1, dec_b1, 2))
    reconstructions = _conv_t(recon, dec_w2, dec_b2, 2)

    recon_loss = jnp.mean((images - reconstructions) ** 2)
    latent = jnp.mean((projected_tokens - features) ** 2)
    loss = latent + COMMITMENT * latent + recon_loss
    return projected_tokens, labels, loss, reconstructions, counts


# speedup vs baseline: 1.0147x; 1.0147x over previous
import jax
import jax.numpy as jnp
from jax import lax
from jax.experimental import pallas as pl

NUM_TOKENS = 8192
CODE_DIM = 256
FEAT_DIM = 192
COMMITMENT = 0.25
N_FLAT = 8 * 56 * 56


def _conv(x, w, b, stride):
    y = lax.conv_general_dilated(x, w, (stride, stride), 'SAME',
                                 dimension_numbers=('NCHW', 'OIHW', 'NCHW'))
    return y + b[None, :, None, None]


def _conv_t(x, w, b, stride):
    y = lax.conv_transpose(x, w, (stride, stride), 'SAME',
                           dimension_numbers=('NCHW', 'OIHW', 'NCHW'))
    return y + b[None, :, None, None]


def _proj_body(tok_ref, wout_ref, b_ref, out_ref):
    out_ref[...] = lax.dot_general(tok_ref[...], wout_ref[...],
                                   (((1,), (1,)), ((), ())),
                                   preferred_element_type=jnp.float32
                                   ) + b_ref[...]


def _proj_out_mm(tokens_flat, wout2d, b2d):
    return pl.pallas_call(
        _proj_body,
        grid=(N_FLAT // 1568,),
        in_specs=[pl.BlockSpec((1568, CODE_DIM), lambda i: (i, 0)),
                  pl.BlockSpec((FEAT_DIM, CODE_DIM), lambda i: (0, 0)),
                  pl.BlockSpec((1, FEAT_DIM), lambda i: (0, 0))],
        out_specs=pl.BlockSpec((1568, FEAT_DIM), lambda i: (i, 0)),
        out_shape=jax.ShapeDtypeStruct((N_FLAT, FEAT_DIM), jnp.float32),
    )(tokens_flat, wout2d, b2d)


def kernel(images, enc_w1, enc_b1, enc_w2, enc_b2, proj_in_w, proj_in_b,
           proj_out_w, proj_out_b, dec_w1, dec_b1, dec_w2, dec_b2, codebook):
    features = jax.nn.relu(_conv(images, enc_w1, enc_b1, 2))
    features = jax.nn.relu(_conv(features, enc_w2, enc_b2, 2))

    x = _conv(features, proj_in_w, proj_in_b, 1)
    inputs_nhwc = jnp.transpose(x, (0, 2, 3, 1))
    flat = inputs_nhwc.reshape(-1, CODE_DIM)
    distances = (jnp.sum(flat ** 2, axis=1, keepdims=True)
                 + jnp.sum(codebook ** 2, axis=1)
                 - 2.0 * jnp.matmul(flat, codebook.T))
    labels = jnp.argmin(distances, axis=1)

    encodings = jax.nn.one_hot(labels, NUM_TOKENS, dtype=flat.dtype)
    quantized = jnp.matmul(encodings, codebook).reshape(inputs_nhwc.shape)
    counts = jnp.bincount(labels, length=NUM_TOKENS)
    quantized = inputs_nhwc + lax.stop_gradient(quantized - inputs_nhwc)
    tokens_flat = quantized.reshape(-1, CODE_DIM)

    proj_flat = _proj_out_mm(tokens_flat,
                             proj_out_w.reshape(FEAT_DIM, CODE_DIM),
                             proj_out_b.reshape(1, FEAT_DIM))
    projected_tokens = jnp.transpose(
        proj_flat.reshape(8, 56, 56, FEAT_DIM), (0, 3, 1, 2))

    recon = jax.nn.relu(_conv_t(projected_tokens, dec_w1, dec_b1, 2))
    reconstructions = _conv_t(recon, dec_w2, dec_b2, 2)

    recon_loss = jnp.mean((images - reconstructions) ** 2)
    latent = jnp.mean((projected_tokens - features) ** 2)
    loss = latent + COMMITMENT * latent + recon_loss
    return projected_tokens, labels, loss, reconstructions, counts


# speedup vs baseline: 1.0180x; 1.0033x over previous
import jax
import jax.numpy as jnp
from jax import lax
from jax.experimental import pallas as pl

NUM_TOKENS = 8192
CODE_DIM = 256
FEAT_DIM = 192
COMMITMENT = 0.25
N_FLAT = 8 * 56 * 56


def _conv(x, w, b, stride):
    y = lax.conv_general_dilated(x, w, (stride, stride), 'SAME',
                                 dimension_numbers=('NCHW', 'OIHW', 'NCHW'))
    return y + b[None, :, None, None]


def _conv_t(x, w, b, stride):
    y = lax.conv_transpose(x, w, (stride, stride), 'SAME',
                           dimension_numbers=('NCHW', 'OIHW', 'NCHW'))
    return y + b[None, :, None, None]


def _proj_body(tok_ref, wout_ref, b_ref, out_ref):
    out_ref[...] = lax.dot_general(tok_ref[...], wout_ref[...],
                                   (((1,), (1,)), ((), ())),
                                   preferred_element_type=jnp.float32
                                   ) + b_ref[...]


def _proj_out_mm(tokens_flat, wout2d, b2d):
    return pl.pallas_call(
        _proj_body,
        grid=(N_FLAT // 3136,),
        in_specs=[pl.BlockSpec((3136, CODE_DIM), lambda i: (i, 0)),
                  pl.BlockSpec((FEAT_DIM, CODE_DIM), lambda i: (0, 0)),
                  pl.BlockSpec((1, FEAT_DIM), lambda i: (0, 0))],
        out_specs=pl.BlockSpec((3136, FEAT_DIM), lambda i: (i, 0)),
        out_shape=jax.ShapeDtypeStruct((N_FLAT, FEAT_DIM), jnp.float32),
    )(tokens_flat, wout2d, b2d)


def kernel(images, enc_w1, enc_b1, enc_w2, enc_b2, proj_in_w, proj_in_b,
           proj_out_w, proj_out_b, dec_w1, dec_b1, dec_w2, dec_b2, codebook):
    features = jax.nn.relu(_conv(images, enc_w1, enc_b1, 2))
    features = jax.nn.relu(_conv(features, enc_w2, enc_b2, 2))

    x = _conv(features, proj_in_w, proj_in_b, 1)
    inputs_nhwc = jnp.transpose(x, (0, 2, 3, 1))
    flat = inputs_nhwc.reshape(-1, CODE_DIM)
    distances = (jnp.sum(flat ** 2, axis=1, keepdims=True)
                 + jnp.sum(codebook ** 2, axis=1)
                 - 2.0 * jnp.matmul(flat, codebook.T))
    labels = jnp.argmin(distances, axis=1)

    encodings = jax.nn.one_hot(labels, NUM_TOKENS, dtype=flat.dtype)
    quantized = jnp.matmul(encodings, codebook).reshape(inputs_nhwc.shape)
    counts = jnp.bincount(labels, length=NUM_TOKENS)
    quantized = inputs_nhwc + lax.stop_gradient(quantized - inputs_nhwc)
    tokens_flat = quantized.reshape(-1, CODE_DIM)

    proj_flat = _proj_out_mm(tokens_flat,
                             proj_out_w.reshape(FEAT_DIM, CODE_DIM),
                             proj_out_b.reshape(1, FEAT_DIM))
    projected_tokens = jnp.transpose(
        proj_flat.reshape(8, 56, 56, FEAT_DIM), (0, 3, 1, 2))

    recon = jax.nn.relu(_conv_t(projected_tokens, dec_w1, dec_b1, 2))
    reconstructions = _conv_t(recon, dec_w2, dec_b2, 2)

    recon_loss = jnp.mean((images - reconstructions) ** 2)
    latent = jnp.mean((projected_tokens - features) ** 2)
    loss = latent + COMMITMENT * latent + recon_loss
    return projected_tokens, labels, loss, reconstructions, counts
